# baseline (speedup 1.0000x reference)
import functools

import jax
import jax.numpy as jnp
from jax import lax
from jax.experimental import pallas as pl
from jax.experimental.pallas import tpu as pltpu

_ANCHORS = ((12.0, 16.0), (19.0, 36.0), (40.0, 28.0), (36.0, 75.0),
            (76.0, 55.0), (72.0, 146.0), (142.0, 110.0), (192.0, 243.0),
            (459.0, 401.0))
_STRIDES = (32, 16, 8)
_AMASKS = ((6, 7, 8), (3, 4, 5), (0, 1, 2))
_NCLS = 80
_NCH = 5 + _NCLS
_M = 20
_B = 16
_BM = _B * _M
_P = 16
_SI = 8


def _prep_kernel(t_ref, o_ref, o2_ref):
    cls = t_ref[0]
    xn = t_ref[1]
    yn = t_ref[2]
    wn = t_ref[3]
    hn = t_ref[4]
    s = cls + xn + yn + wn + hn
    validrow = (s > 0.0).astype(jnp.float32)
    nlabel = jnp.sum(validrow, axis=1, keepdims=True)
    iota_m = lax.broadcasted_iota(jnp.int32, (_B, _M), 1).astype(jnp.float32)
    iota_b = lax.broadcasted_iota(jnp.int32, (_B, _M), 0)
    valid = (iota_m < nlabel).astype(jnp.float32)
    hasl = jnp.where(nlabel > 0.0, 1.0, 0.0) + jnp.zeros((_B, _M), jnp.float32)
    w5 = wn * 512.0
    h5 = hn * 512.0
    best = jnp.zeros((_B, _M), jnp.float32)
    cur = None
    for k in range(9):
        wa, ha = _ANCHORS[k]
        iw = jnp.minimum(w5, wa)
        ih = jnp.minimum(h5, ha)
        en = ((iw > 0.0) & (ih > 0.0)).astype(jnp.float32)
        ai = iw * ih * en
        iou = ai / (w5 * h5 + wa * ha - ai + 1e-16)
        if cur is None:
            cur = iou
        else:
            upd = iou > cur
            best = jnp.where(upd, float(k), best)
            cur = jnp.where(upd, iou, cur)
    a = best - 3.0 * jnp.floor(best / 3.0)
    blayer = jnp.floor(best / 3.0)
    wab = jnp.zeros_like(best)
    hab = jnp.zeros_like(best)
    for k in range(9):
        wab = jnp.where(best == float(k), _ANCHORS[k][0], wab)
        hab = jnp.where(best == float(k), _ANCHORS[k][1], hab)
    twlog = jnp.log(w5 / wab + 1e-16)
    thlog = jnp.log(h5 / hab + 1e-16)
    sc = jnp.sqrt(2.0 - wn * hn)
    o_ref[0] = valid
    o_ref[1] = hasl
    o_ref[2] = a
    o_ref[3] = blayer
    o_ref[4] = twlog
    o_ref[5] = thlog
    o_ref[6] = sc
    o_ref[7] = cls
    o_ref[8] = xn
    o_ref[9] = yn
    o_ref[10] = wn
    o_ref[11] = hn
    for p in range(12, _P):
        o_ref[p] = jnp.zeros((_B, _M), jnp.float32)
    for l in range(3):
        f = float(512 // _STRIDES[l])
        il = jnp.floor(xn * f)
        jl = jnp.floor(yn * f)
        condl = (valid > 0.0) & (blayer == float(2 - l))
        clsact = jnp.zeros((_B, _M), jnp.float32)
        lw = jnp.zeros((_B, _M), jnp.float32)
        for m in range(_M):
            eqc = ((a == a[:, m:m + 1]) & (il == il[:, m:m + 1]) &
                   (jl == jl[:, m:m + 1]))
            eq = eqc & (cls == cls[:, m:m + 1])
            later = iota_m > float(m)
            dup = jnp.max(jnp.where(condl & eq & later, 1.0, 0.0),
                          axis=1, keepdims=True)
            dupc = jnp.max(jnp.where(condl & eqc & later, 1.0, 0.0),
                           axis=1, keepdims=True)
            sel = condl[:, m:m + 1]
            val = jnp.where(sel & (dup < 0.5), 1.0, 0.0)
            vlw = jnp.where(sel & (dupc < 0.5), 1.0, 0.0)
            clsact = jnp.where(iota_m == float(m), val, clsact)
            lw = jnp.where(iota_m == float(m), vlw, lw)
        flat = jl * f + il
        srow = jnp.floor(flat / 128.0)
        lane = flat - srow * 128.0
        o2_ref[l, 0] = iota_b
        o2_ref[l, 1] = a.astype(jnp.int32)
        o2_ref[l, 2] = srow.astype(jnp.int32)
        o2_ref[l, 3] = lane.astype(jnp.int32)
        o2_ref[l, 4] = lw.astype(jnp.int32)
        o2_ref[l, 5] = clsact.astype(jnp.int32)
        o2_ref[l, 6] = cls.astype(jnp.int32)
        o2_ref[l, 7] = jnp.zeros((_B, _M), jnp.int32)


def _layer_kernel(prep_ref, x_ref, o_ref, *, lid, f, S):
    aidx = pl.program_id(1)
    stride = _STRIDES[lid]
    was = [_ANCHORS[k][0] / stride for k in _AMASKS[lid]]
    has = [_ANCHORS[k][1] / stride for k in _AMASKS[lid]]
    af = aidx.astype(jnp.float32)
    wa = jnp.where(aidx == 0, was[0], jnp.where(aidx == 1, was[1], was[2]))
    ha = jnp.where(aidx == 0, has[0], jnp.where(aidx == 1, has[1], has[2]))
    xr = x_ref[0, 0, 0]
    yr = x_ref[0, 0, 1]
    wr = x_ref[0, 0, 2]
    hr = x_ref[0, 0, 3]
    obr = x_ref[0, 0, 4]
    sx = jax.nn.sigmoid(xr)
    sy = jax.nn.sigmoid(yr)
    so = jax.nn.sigmoid(obr)
    flat = (lax.broadcasted_iota(jnp.int32, (S, 128), 0) * 128 +
            lax.broadcasted_iota(jnp.int32, (S, 128), 1))
    iif = (flat % f).astype(jnp.float32)
    jjf = (flat // f).astype(jnp.float32)
    px = sx + iif
    py = sy + jjf
    pw = jnp.exp(wr) * wa
    ph = jnp.exp(hr) * ha
    pa = pw * ph
    phw = pw * 0.5
    phh = ph * 0.5
    mx = jnp.zeros((S, 128), jnp.float32)
    assigned = jnp.zeros((S, 128), jnp.bool_)
    txf = jnp.zeros((S, 128), jnp.float32)
    tyf = jnp.zeros((S, 128), jnp.float32)
    twl = jnp.zeros((S, 128), jnp.float32)
    thl = jnp.zeros((S, 128), jnp.float32)
    scv = jnp.zeros((S, 128), jnp.float32)
    for m in range(_M):
        valid = prep_ref[0, 0, m] > 0.0
        am = prep_ref[0, 2, m]
        bl = prep_ref[0, 3, m]
        twlog = prep_ref[0, 4, m]
        thlog = prep_ref[0, 5, m]
        scm = prep_ref[0, 6, m]
        tx = prep_ref[0, 8, m] * f
        ty = prep_ref[0, 9, m] * f
        tw = prep_ref[0, 10, m] * f
        th = prep_ref[0, 11, m] * f
        hw = tw * 0.5
        hh = th * 0.5
        tlx = jnp.maximum(px - phw, tx - hw)
        brx = jnp.minimum(px + phw, tx + hw)
        tly = jnp.maximum(py - phh, ty - hh)
        bry = jnp.minimum(py + phh, ty + hh)
        en = ((tlx < brx) & (tly < bry)).astype(jnp.float32)
        ai = (brx - tlx) * (bry - tly) * en
        iou = ai / (pa + tw * th - ai + 1e-16)
        mx = jnp.maximum(mx, jnp.where(valid, iou, 0.0))
        im = jnp.floor(tx)
        jm = jnp.floor(ty)
        condm = valid & (bl == float(2 - lid)) & (am == af)
        mv = condm & (iif == im) & (jjf == jm)
        assigned = assigned | mv
        txf = jnp.where(mv, tx - im, txf)
        tyf = jnp.where(mv, ty - jm, tyf)
        twl = jnp.where(mv, twlog, twl)
        thl = jnp.where(mv, thlog, thl)
        scv = jnp.where(mv, scm, scv)
    hasl = prep_ref[0, 1, 0] > 0.0
    asf = assigned.astype(jnp.float32)
    omb = jnp.where(hasl, jnp.where(mx > 0.7, 0.0, 1.0), 1.0)
    om = jnp.where(assigned, 1.0, omb)
    pobj = jnp.clip(so * om, 1e-7, 1.0 - 1e-7)
    lobj = -(asf * jnp.log(pobj) + (1.0 - asf) * jnp.log(1.0 - pobj))
    w2 = scv * scv
    pxc = jnp.clip(sx * asf, 1e-7, 1.0 - 1e-7)
    pyc = jnp.clip(sy * asf, 1e-7, 1.0 - 1e-7)
    txt = txf * asf
    tyt = tyf * asf
    lxy = (-(txt * jnp.log(pxc) + (1.0 - txt) * jnp.log(1.0 - pxc)) * w2
           - (tyt * jnp.log(pyc) + (1.0 - tyt) * jnp.log(1.0 - pyc)) * w2)
    dw = wr * asf * scv - twl * asf * scv
    dh = hr * asf * scv - thl * asf * scv
    lwh = 0.5 * (dw * dw + dh * dh)
    c0 = -jnp.log(1.0 - jnp.clip(jnp.float32(0.0), 1e-7, 1.0 - 1e-7))
    lcls = jnp.where(assigned, 0.0, _NCLS * c0)
    total = jnp.sum(lobj + lxy + lwh + lcls)
    first = (pl.program_id(0) == 0) & (pl.program_id(1) == 0)

    @pl.when(first)
    def _():
        o_ref[...] = jnp.zeros_like(o_ref)

    ri = lax.broadcasted_iota(jnp.int32, (8, 128), 0)
    ci = lax.broadcasted_iota(jnp.int32, (8, 128), 1)
    o_ref[...] += jnp.where((ri == 0) & (ci == 0), total, 0.0)


def _cls_kernel(s_ref, x_ref, o_ref):
    k = pl.program_id(0)
    lane = s_ref[3, k]
    lw = s_ref[4, k]
    act = s_ref[5, k]
    cidx = s_ref[6, k]
    rows = x_ref[0, 0, pl.ds(5, _NCLS), 0, 0]
    lane_m = lax.broadcasted_iota(jnp.int32, (_NCLS, 128), 1) == lane
    vcol = jnp.sum(jnp.where(lane_m, rows, 0.0), axis=1, keepdims=True)
    p = jnp.clip(jax.nn.sigmoid(vcol), 1e-7, 1.0 - 1e-7)
    nl1 = -jnp.log(1.0 - p)
    t_lw = jnp.sum(nl1)
    rowm = lax.broadcasted_iota(jnp.int32, (_NCLS, 1), 0) == cidx
    t_act = jnp.sum(jnp.where(rowm, -jnp.log(p) - nl1, 0.0))
    contrib = (jnp.where(lw > 0, t_lw, 0.0) +
               jnp.where(act > 0, t_act, 0.0))

    @pl.when(k == 0)
    def _():
        o_ref[...] = jnp.zeros_like(o_ref)

    ri = lax.broadcasted_iota(jnp.int32, (8, 128), 0)
    ci = lax.broadcasted_iota(jnp.int32, (8, 128), 1)
    o_ref[...] += jnp.where((ri == 0) & (ci == 0), contrib, 0.0)


def _run_layer(prep, xresh, lid, f, S):
    kern = functools.partial(_layer_kernel, lid=lid, f=f, S=S)
    return pl.pallas_call(
        kern,
        grid=(_B, 3),
        in_specs=[
            pl.BlockSpec((1, _P, _M), lambda b, a: (b, 0, 0)),
            pl.BlockSpec((1, 1, 5, S, 128), lambda b, a: (b, a, 0, 0, 0)),
        ],
        out_specs=pl.BlockSpec((8, 128), lambda b, a: (0, 0)),
        out_shape=jax.ShapeDtypeStruct((8, 128), jnp.float32),
    )(prep, xresh)


def _run_cls(sidx_l, xresh):
    grid_spec = pltpu.PrefetchScalarGridSpec(
        num_scalar_prefetch=1,
        grid=(_BM,),
        in_specs=[
            pl.BlockSpec((1, 1, _NCH, 1, 1, 128),
                         lambda k, s: (s[0, k], s[1, k], 0, s[2, k], 0, 0)),
        ],
        out_specs=pl.BlockSpec((8, 128), lambda k, s: (0, 0)),
    )
    return pl.pallas_call(
        _cls_kernel,
        grid_spec=grid_spec,
        out_shape=jax.ShapeDtypeStruct((8, 128), jnp.float32),
    )(sidx_l, xresh)


def kernel(out0, out1, out2, targets):
    tgt_t = jnp.transpose(targets, (2, 0, 1))
    prep, sidx = pl.pallas_call(
        _prep_kernel,
        out_shape=(jax.ShapeDtypeStruct((_P, _B, _M), jnp.float32),
                   jax.ShapeDtypeStruct((3, _SI, _B, _M), jnp.int32)),
    )(tgt_t)
    prep = jnp.transpose(prep, (1, 0, 2))
    sidx = sidx.reshape(3, _SI, _BM)
    total = jnp.float32(0.0)
    for lid, out in enumerate((out0, out1, out2)):
        f = out.shape[2]
        S = f * f // 128
        xresh = out.reshape(_B, 3, _NCH, S, 128)
        total = total + _run_layer(prep, xresh, lid, f, S)[0, 0]
        xcls = out.reshape(_B, 3, _NCH, S, 1, 128)
        total = total + _run_cls(sidx[lid], xcls)[0, 0]
    return total


# speedup vs baseline: 1.5144x; 1.5144x over previous
import functools

import jax
import jax.numpy as jnp
from jax import lax
from jax.experimental import pallas as pl
from jax.experimental.pallas import tpu as pltpu

_ANCHORS = ((12.0, 16.0), (19.0, 36.0), (40.0, 28.0), (36.0, 75.0),
            (76.0, 55.0), (72.0, 146.0), (142.0, 110.0), (192.0, 243.0),
            (459.0, 401.0))
_STRIDES = (32, 16, 8)
_AMASKS = ((6, 7, 8), (3, 4, 5), (0, 1, 2))
_NCLS = 80
_NCH = 5 + _NCLS
_M = 20
_B = 16
_P = 24


def _prep_kernel(t_ref, o_ref):
    cls = t_ref[0]
    xn = t_ref[1]
    yn = t_ref[2]
    wn = t_ref[3]
    hn = t_ref[4]
    s = cls + xn + yn + wn + hn
    validrow = (s > 0.0).astype(jnp.float32)
    nlabel = jnp.sum(validrow, axis=1, keepdims=True)
    iota_m = lax.broadcasted_iota(jnp.int32, (_B, _M), 1).astype(jnp.float32)
    valid = (iota_m < nlabel).astype(jnp.float32)
    hasl = jnp.where(nlabel > 0.0, 1.0, 0.0) + jnp.zeros((_B, _M), jnp.float32)
    w5 = wn * 512.0
    h5 = hn * 512.0
    best = jnp.zeros((_B, _M), jnp.float32)
    cur = None
    for k in range(9):
        wa, ha = _ANCHORS[k]
        iw = jnp.minimum(w5, wa)
        ih = jnp.minimum(h5, ha)
        en = ((iw > 0.0) & (ih > 0.0)).astype(jnp.float32)
        ai = iw * ih * en
        iou = ai / (w5 * h5 + wa * ha - ai + 1e-16)
        if cur is None:
            cur = iou
        else:
            upd = iou > cur
            best = jnp.where(upd, float(k), best)
            cur = jnp.where(upd, iou, cur)
    a = best - 3.0 * jnp.floor(best / 3.0)
    blayer = jnp.floor(best / 3.0)
    wab = jnp.zeros_like(best)
    hab = jnp.zeros_like(best)
    for k in range(9):
        wab = jnp.where(best == float(k), _ANCHORS[k][0], wab)
        hab = jnp.where(best == float(k), _ANCHORS[k][1], hab)
    twlog = jnp.log(w5 / wab + 1e-16)
    thlog = jnp.log(h5 / hab + 1e-16)
    sc = jnp.sqrt(2.0 - wn * hn)
    o_ref[0] = valid
    o_ref[1] = hasl
    o_ref[2] = a
    o_ref[3] = blayer
    o_ref[4] = twlog
    o_ref[5] = thlog
    o_ref[6] = sc
    o_ref[7] = cls
    o_ref[8] = xn
    o_ref[9] = yn
    o_ref[10] = wn
    o_ref[11] = hn
    for l in range(3):
        f = float(512 // _STRIDES[l])
        il = jnp.floor(xn * f)
        jl = jnp.floor(yn * f)
        condl = (valid > 0.0) & (blayer == float(2 - l))
        clsact = jnp.zeros((_B, _M), jnp.float32)
        lw = jnp.zeros((_B, _M), jnp.float32)
        for m in range(_M):
            eqc = ((a == a[:, m:m + 1]) & (il == il[:, m:m + 1]) &
                   (jl == jl[:, m:m + 1]))
            eq = eqc & (cls == cls[:, m:m + 1])
            later = iota_m > float(m)
            dup = jnp.max(jnp.where(condl & eq & later, 1.0, 0.0),
                          axis=1, keepdims=True)
            dupc = jnp.max(jnp.where(condl & eqc & later, 1.0, 0.0),
                           axis=1, keepdims=True)
            sel = condl[:, m:m + 1]
            val = jnp.where(sel & (dup < 0.5), 1.0, 0.0)
            vlw = jnp.where(sel & (dupc < 0.5), 1.0, 0.0)
            clsact = jnp.where(iota_m == float(m), val, clsact)
            lw = jnp.where(iota_m == float(m), vlw, lw)
        o_ref[12 + l] = clsact
        o_ref[15 + l] = lw
    for p in range(18, _P):
        o_ref[p] = jnp.zeros((_B, _M), jnp.float32)


def _layer_kernel(prep_ref, x_ref, o_ref, csum_ref, *, lid, f):
    aidx = pl.program_id(1)
    stride = _STRIDES[lid]
    was = [_ANCHORS[k][0] / stride for k in _AMASKS[lid]]
    has = [_ANCHORS[k][1] / stride for k in _AMASKS[lid]]
    af = aidx.astype(jnp.float32)
    wa = jnp.where(aidx == 0, was[0], jnp.where(aidx == 1, was[1], was[2]))
    ha = jnp.where(aidx == 0, has[0], jnp.where(aidx == 1, has[1], has[2]))
    xr = x_ref[0, 0]
    yr = x_ref[0, 1]
    wr = x_ref[0, 2]
    hr = x_ref[0, 3]
    obr = x_ref[0, 4]
    sx = jax.nn.sigmoid(xr)
    sy = jax.nn.sigmoid(yr)
    so = jax.nn.sigmoid(obr)
    iif = lax.broadcasted_iota(jnp.int32, (f, f), 1).astype(jnp.float32)
    jjf = lax.broadcasted_iota(jnp.int32, (f, f), 0).astype(jnp.float32)
    px = sx + iif
    py = sy + jjf
    pw = jnp.exp(wr) * wa
    ph = jnp.exp(hr) * ha
    pa = pw * ph
    phw = pw * 0.5
    phh = ph * 0.5
    mx = jnp.zeros((f, f), jnp.float32)
    assigned = jnp.zeros((f, f), jnp.bool_)
    txf = jnp.zeros((f, f), jnp.float32)
    tyf = jnp.zeros((f, f), jnp.float32)
    twl = jnp.zeros((f, f), jnp.float32)
    thl = jnp.zeros((f, f), jnp.float32)
    scv = jnp.zeros((f, f), jnp.float32)
    csum_ref[0] = 0.0
    for m in range(_M):
        valid = prep_ref[0, 0, m] > 0.0
        am = prep_ref[0, 2, m]
        bl = prep_ref[0, 3, m]
        twlog = prep_ref[0, 4, m]
        thlog = prep_ref[0, 5, m]
        scm = prep_ref[0, 6, m]
        tx = prep_ref[0, 8, m] * f
        ty = prep_ref[0, 9, m] * f
        tw = prep_ref[0, 10, m] * f
        th = prep_ref[0, 11, m] * f
        hw = tw * 0.5
        hh = th * 0.5
        tlx = jnp.maximum(px - phw, tx - hw)
        brx = jnp.minimum(px + phw, tx + hw)
        tly = jnp.maximum(py - phh, ty - hh)
        bry = jnp.minimum(py + phh, ty + hh)
        en = ((tlx < brx) & (tly < bry)).astype(jnp.float32)
        ai = (brx - tlx) * (bry - tly) * en
        iou = ai / (pa + tw * th - ai + 1e-16)
        mx = jnp.maximum(mx, jnp.where(valid, iou, 0.0))
        im = jnp.floor(tx)
        jm = jnp.floor(ty)
        condm = valid & (bl == float(2 - lid)) & (am == af)
        mv = condm & (iif == im) & (jjf == jm)
        assigned = assigned | mv
        txf = jnp.where(mv, tx - im, txf)
        tyf = jnp.where(mv, ty - jm, tyf)
        twl = jnp.where(mv, twlog, twl)
        thl = jnp.where(mv, thlog, thl)
        scv = jnp.where(mv, scm, scv)
        lwm = prep_ref[0, 15 + lid, m] > 0.0
        actm = prep_ref[0, 12 + lid, m] > 0.0
        cidx = prep_ref[0, 7, m].astype(jnp.int32)
        jm_i = jm.astype(jnp.int32)
        im_i = im.astype(jnp.int32)

        @pl.when(condm & (lwm | actm))
        def _():
            rows = x_ref[0, pl.ds(5, _NCLS), pl.ds(jm_i, 1), :][:, 0, :]
            lane_m = lax.broadcasted_iota(jnp.int32, (_NCLS, f), 1) == im_i
            vcol = jnp.sum(jnp.where(lane_m, rows, 0.0), axis=1,
                           keepdims=True)
            p = jnp.clip(jax.nn.sigmoid(vcol), 1e-7, 1.0 - 1e-7)
            nl1 = -jnp.log(1.0 - p)
            t_lw = jnp.sum(nl1)
            rowm = lax.broadcasted_iota(jnp.int32, (_NCLS, 1), 0) == cidx
            t_act = jnp.sum(jnp.where(rowm, -jnp.log(p) - nl1, 0.0))
            csum_ref[0] += (jnp.where(lwm, t_lw, 0.0) +
                            jnp.where(actm, t_act, 0.0))

    hasl = prep_ref[0, 1, 0] > 0.0
    asf = assigned.astype(jnp.float32)
    omb = jnp.where(hasl, jnp.where(mx > 0.7, 0.0, 1.0), 1.0)
    om = jnp.where(assigned, 1.0, omb)
    pobj = jnp.clip(so * om, 1e-7, 1.0 - 1e-7)
    lobj = -(asf * jnp.log(pobj) + (1.0 - asf) * jnp.log(1.0 - pobj))
    w2 = scv * scv
    pxc = jnp.clip(sx * asf, 1e-7, 1.0 - 1e-7)
    pyc = jnp.clip(sy * asf, 1e-7, 1.0 - 1e-7)
    txt = txf * asf
    tyt = tyf * asf
    lxy = (-(txt * jnp.log(pxc) + (1.0 - txt) * jnp.log(1.0 - pxc)) * w2
           - (tyt * jnp.log(pyc) + (1.0 - tyt) * jnp.log(1.0 - pyc)) * w2)
    dw = wr * asf * scv - twl * asf * scv
    dh = hr * asf * scv - thl * asf * scv
    lwh = 0.5 * (dw * dw + dh * dh)
    c0 = -jnp.log(1.0 - jnp.clip(jnp.float32(0.0), 1e-7, 1.0 - 1e-7))
    lcls = jnp.where(assigned, 0.0, _NCLS * c0)
    total = jnp.sum(lobj + lxy + lwh + lcls) + csum_ref[0]
    first = (pl.program_id(0) == 0) & (pl.program_id(1) == 0)

    @pl.when(first)
    def _():
        o_ref[...] = jnp.zeros_like(o_ref)

    ri = lax.broadcasted_iota(jnp.int32, (8, 128), 0)
    ci = lax.broadcasted_iota(jnp.int32, (8, 128), 1)
    o_ref[...] += jnp.where((ri == 0) & (ci == 0), total, 0.0)


def _run_layer(prep, x, lid, f):
    kern = functools.partial(_layer_kernel, lid=lid, f=f)
    return pl.pallas_call(
        kern,
        grid=(_B, 3),
        in_specs=[
            pl.BlockSpec((1, _P, _M), lambda b, a: (b, 0, 0)),
            pl.BlockSpec((1, _NCH, f, f), lambda b, a: (b, a, 0, 0)),
        ],
        out_specs=pl.BlockSpec((8, 128), lambda b, a: (0, 0)),
        out_shape=jax.ShapeDtypeStruct((8, 128), jnp.float32),
        scratch_shapes=[pltpu.SMEM((1,), jnp.float32)],
    )(prep, x)


def kernel(out0, out1, out2, targets):
    tgt_t = jnp.transpose(targets, (2, 0, 1))
    prep = pl.pallas_call(
        _prep_kernel,
        out_shape=jax.ShapeDtypeStruct((_P, _B, _M), jnp.float32),
    )(tgt_t)
    prep = jnp.transpose(prep, (1, 0, 2))
    total = jnp.float32(0.0)
    for lid, out in enumerate((out0, out1, out2)):
        f = out.shape[2]
        total = total + _run_layer(prep, out, lid, f)[0, 0]
    return total


# speedup vs baseline: 2.9494x; 1.9476x over previous
import functools

import jax
import jax.numpy as jnp
from jax import lax
from jax.experimental import pallas as pl
from jax.experimental.pallas import tpu as pltpu

_ANCHORS = ((12.0, 16.0), (19.0, 36.0), (40.0, 28.0), (36.0, 75.0),
            (76.0, 55.0), (72.0, 146.0), (142.0, 110.0), (192.0, 243.0),
            (459.0, 401.0))
_STRIDES = (32, 16, 8)
_AMASKS = ((6, 7, 8), (3, 4, 5), (0, 1, 2))
_NCLS = 80
_NCH = 5 + _NCLS
_M = 20
_B = 16
_P = 24


def _prep_kernel(t_ref, o_ref):
    cls = t_ref[0]
    xn = t_ref[1]
    yn = t_ref[2]
    wn = t_ref[3]
    hn = t_ref[4]
    s = cls + xn + yn + wn + hn
    validrow = (s > 0.0).astype(jnp.float32)
    nlabel = jnp.sum(validrow, axis=1, keepdims=True)
    iota_m = lax.broadcasted_iota(jnp.int32, (_B, _M), 1).astype(jnp.float32)
    valid = (iota_m < nlabel).astype(jnp.float32)
    hasl = jnp.where(nlabel > 0.0, 1.0, 0.0) + jnp.zeros((_B, _M), jnp.float32)
    w5 = wn * 512.0
    h5 = hn * 512.0
    best = jnp.zeros((_B, _M), jnp.float32)
    cur = None
    for k in range(9):
        wa, ha = _ANCHORS[k]
        iw = jnp.minimum(w5, wa)
        ih = jnp.minimum(h5, ha)
        en = ((iw > 0.0) & (ih > 0.0)).astype(jnp.float32)
        ai = iw * ih * en
        iou = ai / (w5 * h5 + wa * ha - ai + 1e-16)
        if cur is None:
            cur = iou
        else:
            upd = iou > cur
            best = jnp.where(upd, float(k), best)
            cur = jnp.where(upd, iou, cur)
    a = best - 3.0 * jnp.floor(best / 3.0)
    blayer = jnp.floor(best / 3.0)
    wab = jnp.zeros_like(best)
    hab = jnp.zeros_like(best)
    for k in range(9):
        wab = jnp.where(best == float(k), _ANCHORS[k][0], wab)
        hab = jnp.where(best == float(k), _ANCHORS[k][1], hab)
    twlog = jnp.log(w5 / wab + 1e-16)
    thlog = jnp.log(h5 / hab + 1e-16)
    sc = jnp.sqrt(2.0 - wn * hn)
    o_ref[0] = valid
    o_ref[1] = hasl
    o_ref[2] = a
    o_ref[3] = blayer
    o_ref[4] = twlog
    o_ref[5] = thlog
    o_ref[6] = sc
    o_ref[7] = cls
    o_ref[8] = xn
    o_ref[9] = yn
    o_ref[10] = wn
    o_ref[11] = hn
    for l in range(3):
        f = float(512 // _STRIDES[l])
        il = jnp.floor(xn * f)
        jl = jnp.floor(yn * f)
        condl = (valid > 0.0) & (blayer == float(2 - l))
        clsact = jnp.zeros((_B, _M), jnp.float32)
        lw = jnp.zeros((_B, _M), jnp.float32)
        for m in range(_M):
            eqc = ((a == a[:, m:m + 1]) & (il == il[:, m:m + 1]) &
                   (jl == jl[:, m:m + 1]))
            eq = eqc & (cls == cls[:, m:m + 1])
            later = iota_m > float(m)
            dup = jnp.max(jnp.where(condl & eq & later, 1.0, 0.0),
                          axis=1, keepdims=True)
            dupc = jnp.max(jnp.where(condl & eqc & later, 1.0, 0.0),
                           axis=1, keepdims=True)
            sel = condl[:, m:m + 1]
            val = jnp.where(sel & (dup < 0.5), 1.0, 0.0)
            vlw = jnp.where(sel & (dupc < 0.5), 1.0, 0.0)
            clsact = jnp.where(iota_m == float(m), val, clsact)
            lw = jnp.where(iota_m == float(m), vlw, lw)
        o_ref[12 + l] = clsact
        o_ref[15 + l] = lw
    for p in range(18, _P):
        o_ref[p] = jnp.zeros((_B, _M), jnp.float32)


def _layer_kernel(prep_ref, x_ref, o_ref, csum_ref, *, lid, f):
    stride = _STRIDES[lid]
    was = [_ANCHORS[k][0] / stride for k in _AMASKS[lid]]
    has = [_ANCHORS[k][1] / stride for k in _AMASKS[lid]]
    iif = lax.broadcasted_iota(jnp.int32, (f, f), 1).astype(jnp.float32)
    jjf = lax.broadcasted_iota(jnp.int32, (f, f), 0).astype(jnp.float32)
    sx, sy, so, wrs, hrs = [], [], [], [], []
    px, py, pw, ph, pa, phw, phh = [], [], [], [], [], [], []
    mx, assigned = [], []
    txf, tyf, twl, thl, scv = [], [], [], [], []
    for anc in range(3):
        base = _NCH * anc
        xr = x_ref[0, base + 0]
        yr = x_ref[0, base + 1]
        wr = x_ref[0, base + 2]
        hr = x_ref[0, base + 3]
        obr = x_ref[0, base + 4]
        sx.append(jax.nn.sigmoid(xr))
        sy.append(jax.nn.sigmoid(yr))
        so.append(jax.nn.sigmoid(obr))
        wrs.append(wr)
        hrs.append(hr)
        px.append(sx[anc] + iif)
        py.append(sy[anc] + jjf)
        pw.append(jnp.exp(wr) * was[anc])
        ph.append(jnp.exp(hr) * has[anc])
        pa.append(pw[anc] * ph[anc])
        phw.append(pw[anc] * 0.5)
        phh.append(ph[anc] * 0.5)
        mx.append(jnp.zeros((f, f), jnp.float32))
        assigned.append(jnp.zeros((f, f), jnp.bool_))
        txf.append(jnp.zeros((f, f), jnp.float32))
        tyf.append(jnp.zeros((f, f), jnp.float32))
        twl.append(jnp.zeros((f, f), jnp.float32))
        thl.append(jnp.zeros((f, f), jnp.float32))
        scv.append(jnp.zeros((f, f), jnp.float32))
    csum_ref[0] = 0.0
    for m in range(_M):
        valid = prep_ref[0, 0, m] > 0.0
        am = prep_ref[0, 2, m]
        bl = prep_ref[0, 3, m]
        twlog = prep_ref[0, 4, m]
        thlog = prep_ref[0, 5, m]
        scm = prep_ref[0, 6, m]
        tx = prep_ref[0, 8, m] * f
        ty = prep_ref[0, 9, m] * f
        tw = prep_ref[0, 10, m] * f
        th = prep_ref[0, 11, m] * f
        hw = tw * 0.5
        hh = th * 0.5
        im = jnp.floor(tx)
        jm = jnp.floor(ty)
        onlayer = valid & (bl == float(2 - lid))
        cellm = (iif == im) & (jjf == jm)
        for anc in range(3):
            tlx = jnp.maximum(px[anc] - phw[anc], tx - hw)
            brx = jnp.minimum(px[anc] + phw[anc], tx + hw)
            tly = jnp.maximum(py[anc] - phh[anc], ty - hh)
            bry = jnp.minimum(py[anc] + phh[anc], ty + hh)
            en = ((tlx < brx) & (tly < bry)).astype(jnp.float32)
            ai = (brx - tlx) * (bry - tly) * en
            iou = ai / (pa[anc] + tw * th - ai + 1e-16)
            mx[anc] = jnp.maximum(mx[anc], jnp.where(valid, iou, 0.0))
            condm = onlayer & (am == float(anc))
            mv = condm & cellm
            assigned[anc] = assigned[anc] | mv
            txf[anc] = jnp.where(mv, tx - im, txf[anc])
            tyf[anc] = jnp.where(mv, ty - jm, tyf[anc])
            twl[anc] = jnp.where(mv, twlog, twl[anc])
            thl[anc] = jnp.where(mv, thlog, thl[anc])
            scv[anc] = jnp.where(mv, scm, scv[anc])
        lwm = prep_ref[0, 15 + lid, m] > 0.0
        actm = prep_ref[0, 12 + lid, m] > 0.0
        cidx = prep_ref[0, 7, m].astype(jnp.int32)
        jm_i = jm.astype(jnp.int32)
        im_i = im.astype(jnp.int32)
        am_i = am.astype(jnp.int32)

        @pl.when(onlayer & (lwm | actm))
        def _():
            ch0 = am_i * _NCH + 5
            rows = x_ref[0, pl.ds(ch0, _NCLS), pl.ds(jm_i, 1), :][:, 0, :]
            lane_m = lax.broadcasted_iota(jnp.int32, (_NCLS, f), 1) == im_i
            vcol = jnp.sum(jnp.where(lane_m, rows, 0.0), axis=1,
                           keepdims=True)
            p = jnp.clip(jax.nn.sigmoid(vcol), 1e-7, 1.0 - 1e-7)
            nl1 = -jnp.log(1.0 - p)
            t_lw = jnp.sum(nl1)
            rowm = lax.broadcasted_iota(jnp.int32, (_NCLS, 1), 0) == cidx
            t_act = jnp.sum(jnp.where(rowm, -jnp.log(p) - nl1, 0.0))
            csum_ref[0] += (jnp.where(lwm, t_lw, 0.0) +
                            jnp.where(actm, t_act, 0.0))

    hasl = prep_ref[0, 1, 0] > 0.0
    c0 = -jnp.log(1.0 - jnp.clip(jnp.float32(0.0), 1e-7, 1.0 - 1e-7))
    total = csum_ref[0]
    for anc in range(3):
        asf = assigned[anc].astype(jnp.float32)
        omb = jnp.where(hasl, jnp.where(mx[anc] > 0.7, 0.0, 1.0), 1.0)
        om = jnp.where(assigned[anc], 1.0, omb)
        pobj = jnp.clip(so[anc] * om, 1e-7, 1.0 - 1e-7)
        lobj = -(asf * jnp.log(pobj) + (1.0 - asf) * jnp.log(1.0 - pobj))
        w2 = scv[anc] * scv[anc]
        pxc = jnp.clip(sx[anc] * asf, 1e-7, 1.0 - 1e-7)
        pyc = jnp.clip(sy[anc] * asf, 1e-7, 1.0 - 1e-7)
        txt = txf[anc] * asf
        tyt = tyf[anc] * asf
        lxy = (-(txt * jnp.log(pxc) + (1.0 - txt) * jnp.log(1.0 - pxc)) * w2
               - (tyt * jnp.log(pyc) + (1.0 - tyt) * jnp.log(1.0 - pyc)) * w2)
        dw = wrs[anc] * asf * scv[anc] - twl[anc] * asf * scv[anc]
        dh = hrs[anc] * asf * scv[anc] - thl[anc] * asf * scv[anc]
        lwh = 0.5 * (dw * dw + dh * dh)
        lcls = jnp.where(assigned[anc], 0.0, _NCLS * c0)
        total = total + jnp.sum(lobj + lxy + lwh + lcls)
    ri = lax.broadcasted_iota(jnp.int32, (8, 128), 0)
    ci = lax.broadcasted_iota(jnp.int32, (8, 128), 1)
    o_ref[...] = jnp.where((ri == 0) & (ci == 0), total, 0.0)[None]


def _run_layer(prep, x, lid, f):
    kern = functools.partial(_layer_kernel, lid=lid, f=f)
    return pl.pallas_call(
        kern,
        grid=(_B,),
        in_specs=[
            pl.BlockSpec((1, _P, _M), lambda b: (b, 0, 0),
                         memory_space=pltpu.SMEM),
            pl.BlockSpec((1, 3 * _NCH, f, f), lambda b: (b, 0, 0, 0)),
        ],
        out_specs=pl.BlockSpec((1, 8, 128), lambda b: (b, 0, 0)),
        out_shape=jax.ShapeDtypeStruct((_B, 8, 128), jnp.float32),
        scratch_shapes=[pltpu.SMEM((1,), jnp.float32)],
        compiler_params=pltpu.CompilerParams(
            dimension_semantics=("parallel",)),
    )(prep, x)


def kernel(out0, out1, out2, targets):
    tgt_t = jnp.transpose(targets, (2, 0, 1))
    prep = pl.pallas_call(
        _prep_kernel,
        out_shape=jax.ShapeDtypeStruct((_P, _B, _M), jnp.float32),
    )(tgt_t)
    prep = jnp.transpose(prep, (1, 0, 2))
    total = jnp.float32(0.0)
    for lid, out in enumerate((out0, out1, out2)):
        f = out.shape[2]
        total = total + jnp.sum(_run_layer(prep, out, lid, f)[:, 0, 0])
    return total
